# baseline (speedup 1.0000x reference)
import jax
import jax.numpy as jnp
from jax.experimental import pallas as pl
from jax.experimental.pallas import tpu as pltpu


_END_SPLIT = 4


def _chunks(bsz, n):
    s = _END_SPLIT
    h = n // s
    out = [(0, j * h, h, True) for j in range(s)]
    for b in range(1, bsz - 1):
        out.append((b, 0, n, False))
    out.extend([(bsz - 1, j * h, h, True) for j in range(s)])
    return out


def _body(in_hbm, out_hbm, vin_h, vout_h, vin_f, vout_f, in_sems, out_sems):
    bsz, _, n = in_hbm.shape
    chunks = _chunks(bsz, n)

    def bufs(i):
        hi = fi = 0
        for _, _, _, half in chunks[:i]:
            hi += half
            fi += not half
        if chunks[i][3]:
            return vin_h.at[hi], vout_h.at[hi]
        return vin_f.at[fi], vout_f.at[fi]

    for i, (b, off, sz, _) in enumerate(chunks):
        src, _ = bufs(i)
        pltpu.make_async_copy(
            in_hbm.at[b, :, pl.ds(off, sz)], src, in_sems.at[i]
        ).start()
    for i, (b, off, sz, _) in enumerate(chunks):
        src, dst = bufs(i)
        pltpu.make_async_copy(
            in_hbm.at[b, :, pl.ds(off, sz)], src, in_sems.at[i]
        ).wait()
        dst[...] = src[...].T
        pltpu.make_async_copy(
            dst, out_hbm.at[b, pl.ds(off, sz), :], out_sems.at[i]
        ).start()
    for i, (b, off, sz, _) in enumerate(chunks):
        _, dst = bufs(i)
        pltpu.make_async_copy(
            dst, out_hbm.at[b, pl.ds(off, sz), :], out_sems.at[i]
        ).wait()


def kernel(feat, loc, W, b):
    del loc, W, b
    bsz, c, n = feat.shape
    h = n // _END_SPLIT
    chunks = _chunks(bsz, n)
    nchunk = len(chunks)
    nhalf = sum(1 for ch in chunks if ch[3])
    nfull = nchunk - nhalf
    return pl.pallas_call(
        _body,
        in_specs=[pl.BlockSpec(memory_space=pl.ANY)],
        out_specs=pl.BlockSpec(memory_space=pl.ANY),
        out_shape=jax.ShapeDtypeStruct((bsz, n, c), feat.dtype),
        scratch_shapes=[
            pltpu.VMEM((nhalf, c, h), feat.dtype),
            pltpu.VMEM((nhalf, h, c), feat.dtype),
            pltpu.VMEM((nfull, c, n), feat.dtype),
            pltpu.VMEM((nfull, n, c), feat.dtype),
            pltpu.SemaphoreType.DMA((nchunk,)),
            pltpu.SemaphoreType.DMA((nchunk,)),
        ],
    )(feat)


# speedup vs baseline: 1.0105x; 1.0105x over previous
import jax
import jax.numpy as jnp
from jax.experimental import pallas as pl
from jax.experimental.pallas import tpu as pltpu


_END_SPLIT = 2


def _chunks(bsz, n):
    s = _END_SPLIT
    h = n // s
    out = [(0, j * h, h, True) for j in range(s)]
    for b in range(1, bsz - 1):
        out.append((b, 0, n, False))
    out.extend([(bsz - 1, j * h, h, True) for j in range(s)])
    return out


def _body(in_hbm, out_hbm, vin_h, vout_h, vin_f, vout_f, in_sems, out_sems):
    bsz, _, n = in_hbm.shape
    chunks = _chunks(bsz, n)

    def bufs(i):
        hi = fi = 0
        for _, _, _, half in chunks[:i]:
            hi += half
            fi += not half
        if chunks[i][3]:
            return vin_h.at[hi], vout_h.at[hi]
        return vin_f.at[fi], vout_f.at[fi]

    ahead = 2

    def start_read(i):
        b, off, sz, _ = chunks[i]
        src, _ = bufs(i)
        pltpu.make_async_copy(
            in_hbm.at[b, :, pl.ds(off, sz)], src, in_sems.at[i]
        ).start()

    for i in range(min(ahead, len(chunks))):
        start_read(i)
    for i, (b, off, sz, _) in enumerate(chunks):
        if i + ahead < len(chunks):
            start_read(i + ahead)
        src, dst = bufs(i)
        pltpu.make_async_copy(
            in_hbm.at[b, :, pl.ds(off, sz)], src, in_sems.at[i]
        ).wait()
        dst[...] = src[...].T
        pltpu.make_async_copy(
            dst, out_hbm.at[b, pl.ds(off, sz), :], out_sems.at[i]
        ).start()
    for i, (b, off, sz, _) in enumerate(chunks):
        _, dst = bufs(i)
        pltpu.make_async_copy(
            dst, out_hbm.at[b, pl.ds(off, sz), :], out_sems.at[i]
        ).wait()


def kernel(feat, loc, W, b):
    del loc, W, b
    bsz, c, n = feat.shape
    h = n // _END_SPLIT
    chunks = _chunks(bsz, n)
    nchunk = len(chunks)
    nhalf = sum(1 for ch in chunks if ch[3])
    nfull = nchunk - nhalf
    return pl.pallas_call(
        _body,
        in_specs=[pl.BlockSpec(memory_space=pl.ANY)],
        out_specs=pl.BlockSpec(memory_space=pl.ANY),
        out_shape=jax.ShapeDtypeStruct((bsz, n, c), feat.dtype),
        scratch_shapes=[
            pltpu.VMEM((nhalf, c, h), feat.dtype),
            pltpu.VMEM((nhalf, h, c), feat.dtype),
            pltpu.VMEM((nfull, c, n), feat.dtype),
            pltpu.VMEM((nfull, n, c), feat.dtype),
            pltpu.SemaphoreType.DMA((nchunk,)),
            pltpu.SemaphoreType.DMA((nchunk,)),
        ],
    )(feat)
